# baseline (speedup 1.0000x reference)
import functools

import jax
import jax.numpy as jnp
from jax import lax
from jax.experimental import pallas as pl
from jax.experimental.pallas import tpu as pltpu
from jax.experimental.pallas import tpu_sc as plsc

N = 10000
D = 128
E = 320000

NC = 2
NS = 16
NW = NC * NS
CH = 128
K = 80
E_PAD = NW * CH * K
N_PAD = 10240
ZR = N_PAD // NS
DFR = N_PAD * 16 // 128


_MESH = plsc.VectorSubcoreMesh(core_axis_name="c", subcore_axis_name="s")


def _unpack_row(pk_v, j, idx_v, r, shift, mask):
    for u in range(CH // 16):
        v = pk_v[j, pl.ds(u * 16, 16)]
        if shift:
            v = lax.shift_right_logical(v, shift)
        if mask:
            v = lax.bitwise_and(v, mask)
        idx_v[r, pl.ds(u * 16, 16)] = v


def _sc_segsum_body(h_hbm, pk_hbm, zero_hbm, agg_out,
                    pk_v, idx_v, rows_a, rows_b, agg_sh,
                    sem_a, sem_b, sem_sa, sem_sb):
    c = lax.axis_index("c")
    s = lax.axis_index("s")
    w = c * NS + s
    for r in range(ZR // 128):
        pltpu.sync_copy(zero_hbm, agg_sh.at[pl.ds(s * ZR + r * 128, 128)])
    pltpu.sync_copy(pk_hbm.at[w], pk_v)
    plsc.subcore_barrier()

    _unpack_row(pk_v, 0, idx_v, 0, 0, 16383)
    pltpu.async_copy(h_hbm.at[idx_v.at[0]], rows_a, sem_a)
    _unpack_row(pk_v, 1, idx_v, 1, 0, 16383)
    pltpu.async_copy(h_hbm.at[idx_v.at[1]], rows_b, sem_b)

    def pair(t, carry):
        ja = 2 * t
        pltpu.make_async_copy(h_hbm.at[idx_v.at[0]], rows_a, sem_a).wait()
        _unpack_row(pk_v, ja, idx_v, 2, 14, 0)
        pltpu.async_copy(rows_a, agg_sh.at[idx_v.at[2]], sem_sa, add=True)
        jb = ja + 1
        pltpu.make_async_copy(h_hbm.at[idx_v.at[1]], rows_b, sem_b).wait()
        _unpack_row(pk_v, jb, idx_v, 3, 14, 0)
        pltpu.async_copy(rows_b, agg_sh.at[idx_v.at[3]], sem_sb, add=True)
        pltpu.make_async_copy(rows_a, agg_sh.at[idx_v.at[2]], sem_sa).wait()
        _unpack_row(pk_v, (ja + 2) % K, idx_v, 0, 0, 16383)
        pltpu.async_copy(h_hbm.at[idx_v.at[0]], rows_a, sem_a)
        pltpu.make_async_copy(rows_b, agg_sh.at[idx_v.at[3]], sem_sb).wait()
        _unpack_row(pk_v, (jb + 2) % K, idx_v, 1, 0, 16383)
        pltpu.async_copy(h_hbm.at[idx_v.at[1]], rows_b, sem_b)
        return carry

    lax.fori_loop(0, K // 2, pair, 0)
    pltpu.make_async_copy(h_hbm.at[idx_v.at[0]], rows_a, sem_a).wait()
    pltpu.make_async_copy(h_hbm.at[idx_v.at[1]], rows_b, sem_b).wait()
    plsc.subcore_barrier()
    pltpu.sync_copy(agg_sh.at[pl.ds(s * ZR, ZR)],
                    agg_out.at[c, pl.ds(s * ZR, ZR)])


_sc_segsum_call = pl.kernel(
    _sc_segsum_body,
    out_type=[jax.ShapeDtypeStruct((NC, N_PAD, D), jnp.float32)],
    mesh=_MESH,
    scratch_types=[
        pltpu.VMEM((K, CH), jnp.int32),
        pltpu.VMEM((4, CH), jnp.int32),
        pltpu.VMEM((CH, D), jnp.float32),
        pltpu.VMEM((CH, D), jnp.float32),
        pltpu.VMEM_SHARED((N_PAD, D), jnp.float32),
        pltpu.SemaphoreType.DMA,
        pltpu.SemaphoreType.DMA,
        pltpu.SemaphoreType.DMA,
        pltpu.SemaphoreType.DMA,
    ],
)


def _sc_deg_body(pk_hbm, ones_hbm, zero_hbm, deg_out, pk_v, dst_v, ones_v,
                 deg_sh):
    c = lax.axis_index("c")
    s = lax.axis_index("s")
    w = c * NS + s
    for r in range(ZR // 128):
        pltpu.sync_copy(zero_hbm, deg_sh.at[pl.ds(s * ZR + r * 128, 128)])
    pltpu.sync_copy(ones_hbm, ones_v)
    pltpu.sync_copy(pk_hbm.at[w], pk_v)

    def row(j, carry):
        for u in range(CH // 16):
            v = pk_v[j, pl.ds(u * 16, 16)]
            dst_v[j, pl.ds(u * 16, 16)] = lax.shift_right_logical(v, 14)
        return carry

    lax.fori_loop(0, K, row, 0)
    plsc.subcore_barrier()

    def chunk(j, carry):
        pltpu.sync_copy(ones_v, deg_sh.at[dst_v.at[j]], add=True)
        return carry

    lax.fori_loop(0, K, chunk, 0)
    plsc.subcore_barrier()
    pltpu.sync_copy(deg_sh.at[pl.ds(s * ZR, ZR)],
                    deg_out.at[c, pl.ds(s * ZR, ZR)])


_sc_deg_call = pl.kernel(
    _sc_deg_body,
    out_type=[jax.ShapeDtypeStruct((NC, N_PAD, D), jnp.float32)],
    mesh=_MESH,
    scratch_types=[
        pltpu.VMEM((K, CH), jnp.int32),
        pltpu.VMEM((K, CH), jnp.int32),
        pltpu.VMEM((CH, D), jnp.float32),
        pltpu.VMEM_SHARED((N_PAD, D), jnp.float32),
    ],
)


BLK = 1000


def _tc_layer(parts, deg, h, WlT, WrT, b, relu: bool):
    def body(p_ref, d_ref, h_ref, wl_ref, wr_ref, b_ref, o_ref):
        sum_ = p_ref[0] + p_ref[1]
        dg = d_ref[0, :, 0:1] + d_ref[1, :, 0:1]
        agg = sum_ / jnp.maximum(dg, 1.0)
        acc = jnp.dot(agg, wl_ref[...], preferred_element_type=jnp.float32)
        acc = acc + jnp.dot(h_ref[...], wr_ref[...],
                            preferred_element_type=jnp.float32)
        acc = acc + b_ref[...]
        if relu:
            acc = jnp.maximum(acc, 0.0)
        o_ref[...] = acc

    return pl.pallas_call(
        body,
        grid=(N // BLK,),
        in_specs=[
            pl.BlockSpec((NC, BLK, D), lambda i: (0, i, 0)),
            pl.BlockSpec((NC, BLK, D), lambda i: (0, i, 0)),
            pl.BlockSpec((BLK, D), lambda i: (i, 0)),
            pl.BlockSpec((D, D), lambda i: (0, 0)),
            pl.BlockSpec((D, D), lambda i: (0, 0)),
            pl.BlockSpec((1, D), lambda i: (0, 0)),
        ],
        out_specs=pl.BlockSpec((BLK, D), lambda i: (i, 0)),
        out_shape=jax.ShapeDtypeStruct((N, D), jnp.float32),
    )(parts, deg, h, WlT, WrT, b)


@jax.jit
def kernel(x, edge_index, W1_l, W1_r, b1, W2_l, W2_r, b2):
    src = edge_index[0].astype(jnp.int32)
    dst = edge_index[1].astype(jnp.int32)
    pad = E_PAD - E
    src_p = jnp.concatenate(
        [src, (jnp.arange(pad, dtype=jnp.int32) * 131) % N])
    dst_p = jnp.concatenate(
        [dst, N + (jnp.arange(pad, dtype=jnp.int32) % (N_PAD - N))])
    pk_p = ((dst_p << 14) | src_p).reshape(NW, K, CH)
    zero = jnp.zeros((CH, D), jnp.float32)
    ones_rows = jnp.ones((CH, D), jnp.float32)

    (deg,) = _sc_deg_call(pk_p, ones_rows, zero)
    (agg1,) = _sc_segsum_call(x, pk_p, zero)
    h = _tc_layer(agg1, deg, x, W1_l.T, W1_r.T, b1.reshape(1, D), relu=True)
    (agg2,) = _sc_segsum_call(h, pk_p, zero)
    out = _tc_layer(agg2, deg, h, W2_l.T, W2_r.T, b2.reshape(1, D),
                    relu=False)
    return out


# speedup vs baseline: 1.2017x; 1.2017x over previous
import functools

import jax
import jax.numpy as jnp
from jax import lax
from jax.experimental import pallas as pl
from jax.experimental.pallas import tpu as pltpu
from jax.experimental.pallas import tpu_sc as plsc

N = 10000
D = 128
E = 320000

NC = 2
NS = 16
NW = NC * NS
CH = 128
K = 80
E_PAD = NW * CH * K
N_PAD = 10240
ZR = N_PAD // NS
DFR = N_PAD * 16 // 128


_MESH = plsc.VectorSubcoreMesh(core_axis_name="c", subcore_axis_name="s")


def _unpack_row(pk_v, j, idx_v, r, shift, mask):
    for u in range(CH // 16):
        v = pk_v[j, pl.ds(u * 16, 16)]
        if shift:
            v = lax.shift_right_logical(v, shift)
        if mask:
            v = lax.bitwise_and(v, mask)
        idx_v[r, pl.ds(u * 16, 16)] = v


def _sc_segsum_body(h_hbm, pk_hbm, zero_hbm, agg_out,
                    pk_v, idx_v, rows_a, rows_b, agg_sh, sem_a, sem_b):
    c = lax.axis_index("c")
    s = lax.axis_index("s")
    w = c * NS + s
    for r in range(ZR // 128):
        pltpu.sync_copy(zero_hbm, agg_sh.at[pl.ds(s * ZR + r * 128, 128)])
    pltpu.sync_copy(pk_hbm.at[w], pk_v)
    plsc.subcore_barrier()

    _unpack_row(pk_v, 0, idx_v, 0, 0, 16383)
    pltpu.async_copy(h_hbm.at[idx_v.at[0]], rows_a, sem_a)
    _unpack_row(pk_v, 1, idx_v, 1, 0, 16383)
    pltpu.async_copy(h_hbm.at[idx_v.at[1]], rows_b, sem_b)

    def pair(t, carry):
        ja = 2 * t
        pltpu.make_async_copy(h_hbm.at[idx_v.at[0]], rows_a, sem_a).wait()
        _unpack_row(pk_v, ja, idx_v, 2, 14, 0)
        pltpu.sync_copy(rows_a, agg_sh.at[idx_v.at[2]], add=True)
        _unpack_row(pk_v, (ja + 2) % K, idx_v, 0, 0, 16383)
        pltpu.async_copy(h_hbm.at[idx_v.at[0]], rows_a, sem_a)
        jb = ja + 1
        pltpu.make_async_copy(h_hbm.at[idx_v.at[1]], rows_b, sem_b).wait()
        _unpack_row(pk_v, jb, idx_v, 3, 14, 0)
        pltpu.sync_copy(rows_b, agg_sh.at[idx_v.at[3]], add=True)
        _unpack_row(pk_v, (jb + 2) % K, idx_v, 1, 0, 16383)
        pltpu.async_copy(h_hbm.at[idx_v.at[1]], rows_b, sem_b)
        return carry

    lax.fori_loop(0, K // 2, pair, 0)
    pltpu.make_async_copy(h_hbm.at[idx_v.at[0]], rows_a, sem_a).wait()
    pltpu.make_async_copy(h_hbm.at[idx_v.at[1]], rows_b, sem_b).wait()
    plsc.subcore_barrier()
    pltpu.sync_copy(agg_sh.at[pl.ds(s * ZR, ZR)],
                    agg_out.at[c, pl.ds(s * ZR, ZR)])


_sc_segsum_call = pl.kernel(
    _sc_segsum_body,
    out_type=[jax.ShapeDtypeStruct((NC, N_PAD, D), jnp.float32)],
    mesh=_MESH,
    scratch_types=[
        pltpu.VMEM((K, CH), jnp.int32),
        pltpu.VMEM((4, CH), jnp.int32),
        pltpu.VMEM((CH, D), jnp.float32),
        pltpu.VMEM((CH, D), jnp.float32),
        pltpu.VMEM_SHARED((N_PAD, D), jnp.float32),
        pltpu.SemaphoreType.DMA,
        pltpu.SemaphoreType.DMA,
    ],
)


def _sc_deg_body(pk_hbm, ones_hbm, zero_hbm, deg_out, pk_v, dst_v, ones_v,
                 deg_sh):
    c = lax.axis_index("c")
    s = lax.axis_index("s")
    w = c * NS + s
    for r in range(ZR // 128):
        pltpu.sync_copy(zero_hbm, deg_sh.at[pl.ds(s * ZR + r * 128, 128)])
    pltpu.sync_copy(ones_hbm, ones_v)
    pltpu.sync_copy(pk_hbm.at[w], pk_v)

    def row(j, carry):
        for u in range(CH // 16):
            v = pk_v[j, pl.ds(u * 16, 16)]
            dst_v[j, pl.ds(u * 16, 16)] = lax.shift_right_logical(v, 14)
        return carry

    lax.fori_loop(0, K, row, 0)
    plsc.subcore_barrier()

    def chunk(j, carry):
        pltpu.sync_copy(ones_v, deg_sh.at[dst_v.at[j]], add=True)
        return carry

    lax.fori_loop(0, K, chunk, 0)
    plsc.subcore_barrier()
    pltpu.sync_copy(deg_sh.at[pl.ds(s * ZR, ZR)],
                    deg_out.at[c, pl.ds(s * ZR, ZR)])


_sc_deg_call = pl.kernel(
    _sc_deg_body,
    out_type=[jax.ShapeDtypeStruct((NC, N_PAD, D), jnp.float32)],
    mesh=_MESH,
    scratch_types=[
        pltpu.VMEM((K, CH), jnp.int32),
        pltpu.VMEM((K, CH), jnp.int32),
        pltpu.VMEM((CH, D), jnp.float32),
        pltpu.VMEM_SHARED((N_PAD, D), jnp.float32),
    ],
)


BLK = 1000


def _tc_layer(parts, deg, h, WlT, WrT, b, relu: bool):
    def body(p_ref, d_ref, h_ref, wl_ref, wr_ref, b_ref, o_ref):
        sum_ = p_ref[0] + p_ref[1]
        dg = d_ref[0, :, 0:1] + d_ref[1, :, 0:1]
        agg = sum_ / jnp.maximum(dg, 1.0)
        acc = jnp.dot(agg, wl_ref[...], preferred_element_type=jnp.float32)
        acc = acc + jnp.dot(h_ref[...], wr_ref[...],
                            preferred_element_type=jnp.float32)
        acc = acc + b_ref[...]
        if relu:
            acc = jnp.maximum(acc, 0.0)
        o_ref[...] = acc

    return pl.pallas_call(
        body,
        grid=(N // BLK,),
        in_specs=[
            pl.BlockSpec((NC, BLK, D), lambda i: (0, i, 0)),
            pl.BlockSpec((NC, BLK, D), lambda i: (0, i, 0)),
            pl.BlockSpec((BLK, D), lambda i: (i, 0)),
            pl.BlockSpec((D, D), lambda i: (0, 0)),
            pl.BlockSpec((D, D), lambda i: (0, 0)),
            pl.BlockSpec((1, D), lambda i: (0, 0)),
        ],
        out_specs=pl.BlockSpec((BLK, D), lambda i: (i, 0)),
        out_shape=jax.ShapeDtypeStruct((N, D), jnp.float32),
    )(parts, deg, h, WlT, WrT, b)


@jax.jit
def kernel(x, edge_index, W1_l, W1_r, b1, W2_l, W2_r, b2):
    src = edge_index[0].astype(jnp.int32)
    dst = edge_index[1].astype(jnp.int32)
    pad = E_PAD - E
    src_p = jnp.concatenate(
        [src, (jnp.arange(pad, dtype=jnp.int32) * 131) % N])
    dst_p = jnp.concatenate(
        [dst, N + (jnp.arange(pad, dtype=jnp.int32) % (N_PAD - N))])
    pk_p = ((dst_p << 14) | src_p).reshape(NW, K, CH)
    zero = jnp.zeros((CH, D), jnp.float32)
    ones_rows = jnp.ones((CH, D), jnp.float32)

    (deg,) = _sc_deg_call(pk_p, ones_rows, zero)
    (agg1,) = _sc_segsum_call(x, pk_p, zero)
    h = _tc_layer(agg1, deg, x, W1_l.T, W1_r.T, b1.reshape(1, D), relu=True)
    (agg2,) = _sc_segsum_call(h, pk_p, zero)
    out = _tc_layer(agg2, deg, h, W2_l.T, W2_r.T, b2.reshape(1, D),
                    relu=False)
    return out
